# baseline (speedup 1.0000x reference)
import functools

import jax
import jax.numpy as jnp
from jax import lax
from jax.experimental import pallas as pl
from jax.experimental.pallas import tpu as pltpu
from jax.experimental.pallas import tpu_sc as plsc

NC = 2
NS = 16
NW = NC * NS


def _build(B, S, V, D):
    assert B == 128 * NW and S % 8 == 0 and D % 16 == 0
    ST = S // 8
    DT = D // 8

    mesh = plsc.VectorSubcoreMesh(core_axis_name="c", subcore_axis_name="s")

    @functools.partial(
        pl.kernel,
        mesh=mesh,
        compiler_params=pltpu.CompilerParams(
            use_tc_tiling_on_sc=False, needs_layout_passes=False,
            disable_bounds_checks=True),
        out_type=jax.ShapeDtypeStruct((S, DT, NW, 8, 128), jnp.float32),
        scratch_types=[
            pltpu.VMEM((ST, 8, 128), jnp.int32),
            [pltpu.VMEM((128, D), jnp.float32) for _ in range(4)],
            [pltpu.VMEM((D, 128), jnp.float32) for _ in range(2)],
            pltpu.VMEM((S, D), jnp.float32),
            [pltpu.SemaphoreType.DMA for _ in range(4)],
            [pltpu.SemaphoreType.DMA for _ in range(2)],
            pltpu.SemaphoreType.DMA,
        ],
    )
    def emb(xr_hbm, table_hbm, pos_hbm, y_hbm,
            idx_v, rows_v, tr_v, pos_v, sem_g, sem_o, sem_i):
        cid = lax.axis_index("c")
        sid = lax.axis_index("s")
        w = sid * NC + cid

        d_i = [pltpu.async_copy(xr_hbm.at[st, w], idx_v.at[st], sem_i)
               for st in range(ST)]
        pltpu.sync_copy(pos_hbm, pos_v)
        for d in d_i:
            d.wait()

        def issue_gather(s, par):
            st, sl = s // 8, s % 8
            pltpu.async_copy(
                table_hbm.at[idx_v.at[st, sl]], rows_v[par], sem_g[par])

        def wait_gather(par):
            pltpu.make_async_copy(
                table_hbm.at[idx_v.at[0, 0]], rows_v[par], sem_g[par]).wait()

        def issue_out(s, par):
            for dt in range(DT):
                pltpu.async_copy(
                    tr_v[par].at[pl.ds(dt * 8, 8), :], y_hbm.at[s, dt, w],
                    sem_o[par])

        def wait_out(par):
            for dt in range(DT):
                pltpu.make_async_copy(
                    tr_v[par].at[pl.ds(dt * 8, 8), :], y_hbm.at[0, dt, 0],
                    sem_o[par]).wait()

        def transpose_add(s, par_g, par_t):
            pos_c = [pos_v[s, pl.ds(c * 16, 16)] for c in range(D // 16)]
            d_idx = [jnp.arange(c * 16, c * 16 + 16, dtype=jnp.int32)
                     for c in range(D // 16)]

            @plsc.parallel_loop(0, 128, unroll=4)
            def body(r):
                rsplat = jnp.full((16,), r, dtype=jnp.int32)
                for c in range(D // 16):
                    val = rows_v[par_g][r, pl.ds(c * 16, 16)] + pos_c[c]
                    plsc.store_scatter(tr_v[par_t], [d_idx[c], rsplat], val)

        for s0 in range(3):
            issue_gather(s0, s0)

        def step(sp, carry):
            for j in range(4):
                s = sp * 4 + j
                par_t = j % 2

                if j >= 2:
                    wait_out(par_t)
                else:
                    @pl.when(sp > 0)
                    def _():
                        wait_out(par_t)

                wait_gather(j)

                @pl.when(s + 3 < S)
                def _():
                    issue_gather(s + 3, (j + 3) % 4)

                transpose_add(s, j, par_t)
                issue_out(s, par_t)
            return carry

        lax.fori_loop(0, S // 4, step, 0)
        wait_out(0)
        wait_out(1)

    return emb


def kernel(x, text_table, pos_table):
    B, S = x.shape
    V, D = text_table.shape
    xr = x.astype(jnp.int32).reshape(NW, 128, S // 8, 8).transpose(2, 0, 3, 1)
    emb = _build(B, S, V, D)
    y = emb(xr, text_table, pos_table)
    return jnp.transpose(y, (2, 4, 0, 1, 3)).reshape(B, S, D)


# speedup vs baseline: 3.5352x; 3.5352x over previous
import functools

import jax
import jax.numpy as jnp
from jax import lax
from jax.experimental import pallas as pl
from jax.experimental.pallas import tpu as pltpu
from jax.experimental.pallas import tpu_sc as plsc

NC = 2
NS = 16
NW = NC * NS


def _build(B, S, V, D):
    assert B == 128 * NW and S % 8 == 0 and D % 16 == 0
    ST = S // 8
    DT = D // 8

    mesh = plsc.VectorSubcoreMesh(core_axis_name="c", subcore_axis_name="s")

    @functools.partial(
        pl.kernel,
        mesh=mesh,
        compiler_params=pltpu.CompilerParams(
            use_tc_tiling_on_sc=False, needs_layout_passes=False,
            disable_bounds_checks=True),
        out_type=jax.ShapeDtypeStruct((S, DT, NW, 8, 128), jnp.float32),
        scratch_types=[
            pltpu.VMEM((ST, 8, 128), jnp.int32),
            [pltpu.VMEM((128, D), jnp.float32) for _ in range(4)],
            [pltpu.VMEM((D, 129), jnp.float32) for _ in range(2)],
            pltpu.VMEM((S, D), jnp.float32),
            [pltpu.SemaphoreType.DMA for _ in range(4)],
            [pltpu.SemaphoreType.DMA for _ in range(2)],
            pltpu.SemaphoreType.DMA,
        ],
    )
    def emb(xr_hbm, table_hbm, pos_hbm, y_hbm,
            idx_v, rows_v, tr_v, pos_v, sem_g, sem_o, sem_i):
        cid = lax.axis_index("c")
        sid = lax.axis_index("s")
        w = sid * NC + cid

        d_i = [pltpu.async_copy(xr_hbm.at[st, w], idx_v.at[st], sem_i)
               for st in range(ST)]
        pltpu.sync_copy(pos_hbm, pos_v)
        for d in d_i:
            d.wait()

        def issue_gather(s, par):
            st, sl = s // 8, s % 8
            pltpu.async_copy(
                table_hbm.at[idx_v.at[st, sl]], rows_v[par], sem_g[par])

        def wait_gather(par):
            pltpu.make_async_copy(
                table_hbm.at[idx_v.at[0, 0]], rows_v[par], sem_g[par]).wait()

        def issue_out(s, par):
            for dt in range(DT):
                pltpu.async_copy(
                    tr_v[par].at[pl.ds(dt * 8, 8), pl.ds(0, 128)],
                    y_hbm.at[s, dt, w], sem_o[par])

        def wait_out(par):
            for dt in range(DT):
                pltpu.make_async_copy(
                    tr_v[par].at[pl.ds(dt * 8, 8), pl.ds(0, 128)],
                    y_hbm.at[0, dt, 0], sem_o[par]).wait()

        def transpose_add(s, par_g, par_t):
            pos_c = [pos_v[s, pl.ds(c * 16, 16)] for c in range(D // 16)]
            d_idx = [jnp.arange(c * 16, c * 16 + 16, dtype=jnp.int32)
                     for c in range(D // 16)]

            @plsc.parallel_loop(0, 128, unroll=4)
            def body(r):
                rsplat = jnp.full((16,), r, dtype=jnp.int32)
                for c in range(D // 16):
                    val = rows_v[par_g][r, pl.ds(c * 16, 16)] + pos_c[c]
                    plsc.store_scatter(tr_v[par_t], [d_idx[c], rsplat], val)

        for s0 in range(3):
            issue_gather(s0, s0)

        def step(sp, carry):
            for j in range(4):
                s = sp * 4 + j
                par_t = j % 2

                if j >= 2:
                    wait_out(par_t)
                else:
                    @pl.when(sp > 0)
                    def _():
                        wait_out(par_t)

                wait_gather(j)

                @pl.when(s + 3 < S)
                def _():
                    issue_gather(s + 3, (j + 3) % 4)

                transpose_add(s, j, par_t)
                issue_out(s, par_t)
            return carry

        lax.fori_loop(0, S // 4, step, 0)
        wait_out(0)
        wait_out(1)

    return emb


def kernel(x, text_table, pos_table):
    B, S = x.shape
    V, D = text_table.shape
    xr = x.astype(jnp.int32).reshape(NW, 128, S // 8, 8).transpose(2, 0, 3, 1)
    emb = _build(B, S, V, D)
    y = emb(xr, text_table, pos_table)
    return jnp.transpose(y, (2, 4, 0, 1, 3)).reshape(B, S, D)
